# baseline (speedup 1.0000x reference)
import functools

import jax
import jax.numpy as jnp
from jax import lax
from jax.experimental import pallas as pl
from jax.experimental.pallas import tpu as pltpu
from jax.experimental.pallas import tpu_sc as plsc

_N = 10000
_E = 160000
_D_IN = 128
_HID = 400
_D_OUT = 4

_NC = 2
_NS = 16
_L = 256
_EPAD = 163840
_EROWS = _EPAD // _L
_ROWS_F = _EROWS // _NS
_ROWS_E = _EROWS // (_NC * _NS)
_DH = 80
_NP1 = _N + 1
_DZ = 8
_SLICE = _N // _NS

_R = 1000
_HIDP = 512


def _sc_mesh():
    return plsc.VectorSubcoreMesh(core_axis_name="c", subcore_axis_name="s")


def _agg_pipeline(t_h, idxs, idxd, acc, rs, semg, sems, nch):
    def gather(kk, b):
        pltpu.async_copy(t_h.at[idxs.at[kk]], rs[b], semg[b])

    def wait_scatter(kk, b):
        pltpu.make_async_copy(rs[b], acc.at[idxd.at[kk]], sems[b]).wait()

    gather(0, 0)
    gather(1, 1)
    nj = nch // 4

    def body(j, carry):
        for i in range(4):
            kk = 4 * j + i
            bn = (i + 2) % 4
            pltpu.make_async_copy(t_h.at[idxs.at[kk]], rs[i], semg[i]).wait()
            pltpu.async_copy(rs[i], acc.at[idxd.at[kk]], sems[i], add=True)
            if i < 2:
                @pl.when(j > 0)
                def _():
                    wait_scatter(kk - 2, bn)
                gather(kk + 2, bn)
            else:
                wait_scatter(kk - 2, bn)

                @pl.when(j < nj - 1)
                def _():
                    gather(kk + 2, bn)
        return carry

    lax.fori_loop(0, nj, body, 0)
    wait_scatter(nch - 2, 2)
    wait_scatter(nch - 1, 3)


def _agg_scratch(rows, d, dt):
    return [
        pltpu.VMEM((rows, _L), jnp.int32),
        pltpu.VMEM((rows, _L), jnp.int32),
        pltpu.VMEM((_L, d), dt),
        pltpu.VMEM((_L, d), dt),
        pltpu.VMEM((_L, d), dt),
        pltpu.VMEM((_L, d), dt),
        pltpu.VMEM_SHARED((_N, d), dt),
        pltpu.SemaphoreType.DMA,
        pltpu.SemaphoreType.DMA,
        pltpu.SemaphoreType.DMA,
        pltpu.SemaphoreType.DMA,
        pltpu.SemaphoreType.DMA,
        pltpu.SemaphoreType.DMA,
        pltpu.SemaphoreType.DMA,
        pltpu.SemaphoreType.DMA,
    ]


def _make_sc_agg_feat():
    @functools.partial(
        pl.kernel,
        out_type=jax.ShapeDtypeStruct((_NC, _N, _DH), jnp.bfloat16),
        mesh=_sc_mesh(),
        compiler_params=pltpu.CompilerParams(use_tc_tiling_on_sc=False),
        scratch_types=_agg_scratch(_ROWS_F, _DH, jnp.bfloat16),
    )
    def k(t_h, src_h, dst_h, zrow_h, acc_out, idxs, idxd,
          r0, r1, r2, r3, acc, g0, g1, g2, g3, s0, s1, s2, s3):
        c = lax.axis_index("c")
        s = lax.axis_index("s")
        pltpu.sync_copy(zrow_h, acc.at[pl.ds(s * _SLICE, _SLICE)])
        base = s * _ROWS_F
        pltpu.sync_copy(src_h.at[c, pl.ds(base, _ROWS_F)], idxs)
        pltpu.sync_copy(dst_h.at[pl.ds(base, _ROWS_F)], idxd)
        plsc.subcore_barrier()
        _agg_pipeline(t_h, idxs, idxd, acc, (r0, r1, r2, r3),
                      (g0, g1, g2, g3), (s0, s1, s2, s3), _ROWS_F)
        plsc.subcore_barrier()
        pltpu.sync_copy(acc.at[pl.ds(s * _SLICE, _SLICE)],
                        acc_out.at[c, pl.ds(s * _SLICE, _SLICE)])

    return k


def _make_sc_agg_edge():
    @functools.partial(
        pl.kernel,
        out_type=jax.ShapeDtypeStruct((_NC, _N, _DZ), jnp.float32),
        mesh=_sc_mesh(),
        compiler_params=pltpu.CompilerParams(use_tc_tiling_on_sc=False),
        scratch_types=_agg_scratch(_ROWS_E, _DZ, jnp.float32),
    )
    def k(t_h, src_h, dst_h, zrow_h, acc_out, idxs, idxd,
          r0, r1, r2, r3, acc, g0, g1, g2, g3, s0, s1, s2, s3):
        c = lax.axis_index("c")
        s = lax.axis_index("s")
        pltpu.sync_copy(zrow_h, acc.at[pl.ds(s * _SLICE, _SLICE)])
        base = c * (_EROWS // _NC) + s * _ROWS_E
        pltpu.sync_copy(src_h.at[pl.ds(base, _ROWS_E)], idxs)
        pltpu.sync_copy(dst_h.at[pl.ds(base, _ROWS_E)], idxd)
        plsc.subcore_barrier()
        _agg_pipeline(t_h, idxs, idxd, acc, (r0, r1, r2, r3),
                      (g0, g1, g2, g3), (s0, s1, s2, s3), _ROWS_E)
        plsc.subcore_barrier()
        pltpu.sync_copy(acc.at[pl.ds(s * _SLICE, _SLICE)],
                        acc_out.at[c, pl.ds(s * _SLICE, _SLICE)])

    return k


def _fuse1_body(pacc_ref, x_ref, w1_ref, b1_ref, w2_ref, z_ref, deg_ref):
    agg = jnp.concatenate(
        [pacc_ref[0], pacc_ref[1, :, 0:_D_IN - _DH]],
        axis=1).astype(jnp.float32) + x_ref[...]
    deg = pacc_ref[1, :, _D_IN - _DH:_D_IN - _DH + 1].astype(jnp.float32) + 1.0
    mean = agg / deg
    h = jnp.maximum(
        jnp.dot(mean, w1_ref[...], preferred_element_type=jnp.float32)
        + b1_ref[...], 0.0)
    z = jnp.dot(h, w2_ref[...], preferred_element_type=jnp.float32)
    z_ref[...] = z
    deg_ref[...] = jnp.broadcast_to(deg, deg_ref.shape)


def _fuse1(pacc, x, w1p, b1p, w2p):
    return pl.pallas_call(
        _fuse1_body,
        grid=(_N // _R,),
        in_specs=[
            pl.BlockSpec((2, _R, _DH), lambda i: (0, i, 0)),
            pl.BlockSpec((_R, _D_IN), lambda i: (i, 0)),
            pl.BlockSpec((_D_IN, _HIDP), lambda i: (0, 0)),
            pl.BlockSpec((1, _HIDP), lambda i: (0, 0)),
            pl.BlockSpec((_HIDP, _DZ), lambda i: (0, 0)),
        ],
        out_specs=[
            pl.BlockSpec((_R, _DZ), lambda i: (i, 0)),
            pl.BlockSpec((_R, _DZ), lambda i: (i, 0)),
        ],
        out_shape=[
            jax.ShapeDtypeStruct((_N, _DZ), jnp.float32),
            jax.ShapeDtypeStruct((_N, _DZ), jnp.float32),
        ],
    )(pacc, x, w1p, b1p, w2p)


def _fuse2_body(q_ref, z_ref, deg_ref, b2_ref, o_ref):
    ssum = q_ref[0] + q_ref[1] + z_ref[...]
    o = jnp.maximum(ssum / deg_ref[...] + b2_ref[...], 0.0)
    o_ref[...] = o[:, 0:_D_OUT]


def _fuse2(q, z, deg, b2p):
    return pl.pallas_call(
        _fuse2_body,
        grid=(_N // _R,),
        in_specs=[
            pl.BlockSpec((2, _R, _DZ), lambda i: (0, i, 0)),
            pl.BlockSpec((_R, _DZ), lambda i: (i, 0)),
            pl.BlockSpec((_R, _DZ), lambda i: (i, 0)),
            pl.BlockSpec((1, _DZ), lambda i: (0, 0)),
        ],
        out_specs=pl.BlockSpec((_R, _D_OUT), lambda i: (i, 0)),
        out_shape=jax.ShapeDtypeStruct((_N, _D_OUT), jnp.float32),
    )(q, z, deg, b2p)


@jax.jit
def kernel(x, edge_index, W1, U1, c1, b1, W2, U2, c2, b2):
    src = edge_index[0]
    dst = edge_index[1]
    pad = _EPAD - _E
    gsrc = jnp.where(src != dst, src, _N)
    gsrc_p = jnp.concatenate([gsrc, jnp.full((pad,), _N, jnp.int32)])
    dst_p = jnp.concatenate([dst, jnp.zeros((pad,), jnp.int32)])
    src2d = gsrc_p.reshape(_EROWS, _L)
    dst2d = dst_p.reshape(_EROWS, _L)
    src_stk = jnp.stack([src2d, src2d + _NP1])

    xb = x.astype(jnp.bfloat16)
    xaug = jnp.concatenate(
        [xb, jnp.ones((_N, 1), jnp.bfloat16),
         jnp.zeros((_N, 2 * _DH - _D_IN - 1), jnp.bfloat16)], axis=1)
    xaug = jnp.concatenate(
        [xaug, jnp.zeros((1, 2 * _DH), jnp.bfloat16)], axis=0)
    xstk = jnp.concatenate([xaug[:, 0:_DH], xaug[:, _DH:2 * _DH]], axis=0)

    zrow_a = jnp.zeros((_SLICE, _DH), jnp.bfloat16)
    zrow_z = jnp.zeros((_SLICE, _DZ), jnp.float32)

    w1p = jnp.pad(W1, ((0, 0), (0, _HIDP - _HID)))
    b1p = jnp.pad(b1, (0, _HIDP - _HID)).reshape(1, _HIDP)
    w2p = jnp.pad(W2, ((0, _HIDP - _HID), (0, _DZ - _D_OUT)))
    b2p = jnp.pad(b2, (0, _DZ - _D_OUT)).reshape(1, _DZ)

    pacc = _make_sc_agg_feat()(xstk, src_stk, dst2d, zrow_a)
    z, deg = _fuse1(pacc, x, w1p, b1p, w2p)
    zt = jnp.concatenate([z, jnp.zeros((1, _DZ), jnp.float32)], axis=0)
    q = _make_sc_agg_edge()(zt, src2d, dst2d, zrow_z)
    return _fuse2(q, z, deg, b2p)


# speedup vs baseline: 1.0541x; 1.0541x over previous
import functools

import jax
import jax.numpy as jnp
from jax import lax
from jax.experimental import pallas as pl
from jax.experimental.pallas import tpu as pltpu
from jax.experimental.pallas import tpu_sc as plsc

_N = 10000
_E = 160000
_D_IN = 128
_HID = 400
_D_OUT = 4

_NC = 2
_NS = 16
_L = 128
_EPAD = 163840
_EROWS = _EPAD // _L
_ROWS_F = _EROWS // _NS
_ROWS_E = _EROWS // (_NC * _NS)
_DH = 80
_NP1 = _N + 1
_DZ = 8
_SLICE = _N // _NS

_R = 1000
_HIDP = 512


def _sc_mesh():
    return plsc.VectorSubcoreMesh(core_axis_name="c", subcore_axis_name="s")


def _agg_pipeline(t_h, idxs, idxd, acc, rs, semg, sems, nch):
    def gather(kk, b):
        pltpu.async_copy(t_h.at[idxs.at[kk]], rs[b], semg[b])

    def wait_scatter(kk, b):
        pltpu.make_async_copy(rs[b], acc.at[idxd.at[kk]], sems[b]).wait()

    gather(0, 0)
    gather(1, 1)
    nj = nch // 4

    def body(j, carry):
        for i in range(4):
            kk = 4 * j + i
            bn = (i + 2) % 4
            pltpu.make_async_copy(t_h.at[idxs.at[kk]], rs[i], semg[i]).wait()
            pltpu.async_copy(rs[i], acc.at[idxd.at[kk]], sems[i], add=True)
            if i < 2:
                @pl.when(j > 0)
                def _():
                    wait_scatter(kk - 2, bn)
                gather(kk + 2, bn)
            else:
                wait_scatter(kk - 2, bn)

                @pl.when(j < nj - 1)
                def _():
                    gather(kk + 2, bn)
        return carry

    lax.fori_loop(0, nj, body, 0)
    wait_scatter(nch - 2, 2)
    wait_scatter(nch - 1, 3)


def _agg_scratch(rows, d, dt):
    return [
        pltpu.VMEM((rows, _L), jnp.int32),
        pltpu.VMEM((rows, _L), jnp.int32),
        pltpu.VMEM((_L, d), dt),
        pltpu.VMEM((_L, d), dt),
        pltpu.VMEM((_L, d), dt),
        pltpu.VMEM((_L, d), dt),
        pltpu.VMEM_SHARED((_N, d), dt),
        pltpu.SemaphoreType.DMA,
        pltpu.SemaphoreType.DMA,
        pltpu.SemaphoreType.DMA,
        pltpu.SemaphoreType.DMA,
        pltpu.SemaphoreType.DMA,
        pltpu.SemaphoreType.DMA,
        pltpu.SemaphoreType.DMA,
        pltpu.SemaphoreType.DMA,
    ]


def _make_sc_agg_feat():
    @functools.partial(
        pl.kernel,
        out_type=jax.ShapeDtypeStruct((_NC, _N, _DH), jnp.bfloat16),
        mesh=_sc_mesh(),
        compiler_params=pltpu.CompilerParams(use_tc_tiling_on_sc=False),
        scratch_types=_agg_scratch(_ROWS_F, _DH, jnp.bfloat16),
    )
    def k(t_h, src_h, dst_h, zrow_h, acc_out, idxs, idxd,
          r0, r1, r2, r3, acc, g0, g1, g2, g3, s0, s1, s2, s3):
        c = lax.axis_index("c")
        s = lax.axis_index("s")
        pltpu.sync_copy(zrow_h, acc.at[pl.ds(s * _SLICE, _SLICE)])
        base = s * _ROWS_F
        pltpu.sync_copy(src_h.at[c, pl.ds(base, _ROWS_F)], idxs)
        pltpu.sync_copy(dst_h.at[pl.ds(base, _ROWS_F)], idxd)
        plsc.subcore_barrier()
        _agg_pipeline(t_h, idxs, idxd, acc, (r0, r1, r2, r3),
                      (g0, g1, g2, g3), (s0, s1, s2, s3), _ROWS_F)
        plsc.subcore_barrier()
        pltpu.sync_copy(acc.at[pl.ds(s * _SLICE, _SLICE)],
                        acc_out.at[c, pl.ds(s * _SLICE, _SLICE)])

    return k


def _make_sc_agg_edge():
    @functools.partial(
        pl.kernel,
        out_type=jax.ShapeDtypeStruct((_NC, _N, _DZ), jnp.float32),
        mesh=_sc_mesh(),
        compiler_params=pltpu.CompilerParams(use_tc_tiling_on_sc=False),
        scratch_types=_agg_scratch(_ROWS_E, _DZ, jnp.float32),
    )
    def k(t_h, src_h, dst_h, zrow_h, acc_out, idxs, idxd,
          r0, r1, r2, r3, acc, g0, g1, g2, g3, s0, s1, s2, s3):
        c = lax.axis_index("c")
        s = lax.axis_index("s")
        pltpu.sync_copy(zrow_h, acc.at[pl.ds(s * _SLICE, _SLICE)])
        base = c * (_EROWS // _NC) + s * _ROWS_E
        pltpu.sync_copy(src_h.at[pl.ds(base, _ROWS_E)], idxs)
        pltpu.sync_copy(dst_h.at[pl.ds(base, _ROWS_E)], idxd)
        plsc.subcore_barrier()
        _agg_pipeline(t_h, idxs, idxd, acc, (r0, r1, r2, r3),
                      (g0, g1, g2, g3), (s0, s1, s2, s3), _ROWS_E)
        plsc.subcore_barrier()
        pltpu.sync_copy(acc.at[pl.ds(s * _SLICE, _SLICE)],
                        acc_out.at[c, pl.ds(s * _SLICE, _SLICE)])

    return k


def _fuse1_body(pacc_ref, x_ref, w1_ref, b1_ref, w2_ref, z_ref, deg_ref):
    agg = jnp.concatenate(
        [pacc_ref[0], pacc_ref[1, :, 0:_D_IN - _DH]],
        axis=1).astype(jnp.float32) + x_ref[...]
    deg = pacc_ref[1, :, _D_IN - _DH:_D_IN - _DH + 1].astype(jnp.float32) + 1.0
    mean = agg / deg
    h = jnp.maximum(
        jnp.dot(mean, w1_ref[...], preferred_element_type=jnp.float32)
        + b1_ref[...], 0.0)
    z = jnp.dot(h, w2_ref[...], preferred_element_type=jnp.float32)
    z_ref[...] = z
    deg_ref[...] = jnp.broadcast_to(deg, deg_ref.shape)


def _fuse1(pacc, x, w1p, b1p, w2p):
    return pl.pallas_call(
        _fuse1_body,
        grid=(_N // _R,),
        in_specs=[
            pl.BlockSpec((2, _R, _DH), lambda i: (0, i, 0)),
            pl.BlockSpec((_R, _D_IN), lambda i: (i, 0)),
            pl.BlockSpec((_D_IN, _HIDP), lambda i: (0, 0)),
            pl.BlockSpec((1, _HIDP), lambda i: (0, 0)),
            pl.BlockSpec((_HIDP, _DZ), lambda i: (0, 0)),
        ],
        out_specs=[
            pl.BlockSpec((_R, _DZ), lambda i: (i, 0)),
            pl.BlockSpec((_R, _DZ), lambda i: (i, 0)),
        ],
        out_shape=[
            jax.ShapeDtypeStruct((_N, _DZ), jnp.float32),
            jax.ShapeDtypeStruct((_N, _DZ), jnp.float32),
        ],
    )(pacc, x, w1p, b1p, w2p)


def _fuse2_body(q_ref, z_ref, deg_ref, b2_ref, o_ref):
    ssum = q_ref[0] + q_ref[1] + z_ref[...]
    o = jnp.maximum(ssum / deg_ref[...] + b2_ref[...], 0.0)
    o_ref[...] = o[:, 0:_D_OUT]


def _fuse2(q, z, deg, b2p):
    return pl.pallas_call(
        _fuse2_body,
        grid=(_N // _R,),
        in_specs=[
            pl.BlockSpec((2, _R, _DZ), lambda i: (0, i, 0)),
            pl.BlockSpec((_R, _DZ), lambda i: (i, 0)),
            pl.BlockSpec((_R, _DZ), lambda i: (i, 0)),
            pl.BlockSpec((1, _DZ), lambda i: (0, 0)),
        ],
        out_specs=pl.BlockSpec((_R, _D_OUT), lambda i: (i, 0)),
        out_shape=jax.ShapeDtypeStruct((_N, _D_OUT), jnp.float32),
    )(q, z, deg, b2p)


@jax.jit
def kernel(x, edge_index, W1, U1, c1, b1, W2, U2, c2, b2):
    src = edge_index[0]
    dst = edge_index[1]
    pad = _EPAD - _E
    gsrc = jnp.where(src != dst, src, _N)
    gsrc_p = jnp.concatenate([gsrc, jnp.full((pad,), _N, jnp.int32)])
    dst_p = jnp.concatenate([dst, jnp.zeros((pad,), jnp.int32)])
    src2d = gsrc_p.reshape(_EROWS, _L)
    dst2d = dst_p.reshape(_EROWS, _L)
    src_stk = jnp.stack([src2d, src2d + _NP1])

    xb = x.astype(jnp.bfloat16)
    xaug = jnp.concatenate(
        [xb, jnp.ones((_N, 1), jnp.bfloat16),
         jnp.zeros((_N, 2 * _DH - _D_IN - 1), jnp.bfloat16)], axis=1)
    xaug = jnp.concatenate(
        [xaug, jnp.zeros((1, 2 * _DH), jnp.bfloat16)], axis=0)
    xstk = jnp.concatenate([xaug[:, 0:_DH], xaug[:, _DH:2 * _DH]], axis=0)

    zrow_a = jnp.zeros((_SLICE, _DH), jnp.bfloat16)
    zrow_z = jnp.zeros((_SLICE, _DZ), jnp.float32)

    w1p = jnp.pad(W1, ((0, 0), (0, _HIDP - _HID)))
    b1p = jnp.pad(b1, (0, _HIDP - _HID)).reshape(1, _HIDP)
    w2p = jnp.pad(W2, ((0, _HIDP - _HID), (0, _DZ - _D_OUT)))
    b2p = jnp.pad(b2, (0, _DZ - _D_OUT)).reshape(1, _DZ)

    pacc = _make_sc_agg_feat()(xstk, src_stk, dst2d, zrow_a)
    z, deg = _fuse1(pacc, x, w1p, b1p, w2p)
    zt = jnp.concatenate([z, jnp.zeros((1, _DZ), jnp.float32)], axis=0)
    q = _make_sc_agg_edge()(zt, src2d, dst2d, zrow_z)
    return _fuse2(q, z, deg, b2p)
